# baseline (speedup 1.0000x reference)
import functools

import jax
import jax.numpy as jnp
from jax import lax
from jax.experimental import pallas as pl
from jax.experimental.pallas import tpu as pltpu
from jax.experimental.pallas import tpu_sc as plsc

NC = 2
NS = 16
LN = 16
NW = NC * NS

CH = 96
RING = 2


def _tc_encode(x, W_enc, b_enc):
    def body(x_ref, w_ref, b_ref, o_ref):
        o_ref[...] = (
            jnp.dot(x_ref[...], w_ref[...], preferred_element_type=jnp.float32)
            + b_ref[...]
        )

    return pl.pallas_call(
        body,
        out_shape=jax.ShapeDtypeStruct(x.shape, jnp.float32),
    )(x, W_enc, b_enc.reshape(1, -1))


def _tc_node_mlp(p, h, beta, W1a, W1b, b1):
    n, d = h.shape
    hid = W1a.shape[1]

    def body(p_ref, h_ref, beta_ref, wa_ref, wb_ref, b1_ref, ab_ref):
        ne = jnp.maximum(p_ref[0] + p_ref[1] + beta_ref[0] * h_ref[...], 0.0)
        a = jnp.dot(ne, wa_ref[...], preferred_element_type=jnp.float32) + b1_ref[...]
        b = jnp.dot(ne, wb_ref[...], preferred_element_type=jnp.float32)
        ab_ref[...] = jnp.concatenate([a, b], axis=1)

    return pl.pallas_call(
        body,
        in_specs=[
            pl.BlockSpec(memory_space=pltpu.VMEM),
            pl.BlockSpec(memory_space=pltpu.VMEM),
            pl.BlockSpec(memory_space=pltpu.SMEM),
            pl.BlockSpec(memory_space=pltpu.VMEM),
            pl.BlockSpec(memory_space=pltpu.VMEM),
            pl.BlockSpec(memory_space=pltpu.VMEM),
        ],
        out_shape=jax.ShapeDtypeStruct((n, 2 * hid), jnp.float32),
    )(p, h, beta, W1a, W1b, b1.reshape(1, -1))


def _sc_aggregate(h, src_all, didx_all, ew_all, zeros, nch):
    n, d = h.shape
    epw = src_all.shape[0] // NW

    mesh = plsc.VectorSubcoreMesh(core_axis_name="c", subcore_axis_name="s")

    @functools.partial(
        pl.kernel,
        out_type=jax.ShapeDtypeStruct((NC, n, d), jnp.float32),
        mesh=mesh,
        compiler_params=pltpu.CompilerParams(needs_layout_passes=False),
        scratch_types=[
            pltpu.VMEM_SHARED((n, d), jnp.float32),
        ]
        + [pltpu.VMEM((CH,), jnp.int32) for _ in range(RING)]
        + [pltpu.VMEM((CH,), jnp.int32) for _ in range(RING)]
        + [pltpu.VMEM((CH,), jnp.float32) for _ in range(RING)]
        + [pltpu.VMEM((CH, d), jnp.float32) for _ in range(RING)]
        + [pltpu.SemaphoreType.DMA for _ in range(RING)],
    )
    def k(h_hbm, src_hbm, didx_hbm, ew_hbm, z_hbm, part_hbm, acc_sh, *bufs):
        sv = bufs[0:RING]
        dv = bufs[RING:2 * RING]
        ev = bufs[2 * RING:3 * RING]
        rows = bufs[3 * RING:4 * RING]
        gs = bufs[4 * RING:5 * RING]
        c_ax = lax.axis_index("c")
        s_ax = lax.axis_index("s")
        wid = c_ax * NS + s_ax
        slab = n // NS
        row0 = s_ax * slab
        pltpu.sync_copy(z_hbm.at[pl.ds(row0, slab)],
                        acc_sh.at[pl.ds(row0, slab)])
        plsc.subcore_barrier()

        def issue(cc, b):
            base = wid * epw + cc * CH
            pltpu.sync_copy(src_hbm.at[pl.ds(base, CH)], sv[b])
            pltpu.sync_copy(didx_hbm.at[pl.ds(base, CH)], dv[b])
            pltpu.sync_copy(ew_hbm.at[pl.ds(base, CH)], ev[b])
            pltpu.async_copy(h_hbm.at[sv[b]], rows[b], gs[b])

        issue(0, 0)
        issue(1, 1)

        def outer(g, carry):
            for b in range(RING):
                cc = g * RING + b
                pltpu.make_async_copy(h_hbm.at[sv[b]], rows[b],
                                      gs[b]).wait()

                def scale(i, _):
                    splat = jnp.zeros((LN,), jnp.int32) + i
                    w = plsc.load_gather(ev[b], [splat])
                    for r in range(d // LN):
                        rows[b][i, pl.ds(r * LN, LN)] = (
                            rows[b][i, pl.ds(r * LN, LN)] * w
                        )
                    return _

                lax.fori_loop(0, CH, scale, 0, unroll=2)
                pltpu.sync_copy(rows[b], acc_sh.at[dv[b]], add=True)

                @pl.when(cc + 2 < nch)
                def _():
                    issue(cc + 2, b)
            return carry

        lax.fori_loop(0, nch // RING, outer, 0)
        plsc.subcore_barrier()
        pltpu.sync_copy(acc_sh.at[pl.ds(row0, slab)],
                        part_hbm.at[c_ax, pl.ds(row0, slab)])

    return k(h, src_all, didx_all, ew_all, zeros)


def _sc_edge_logits(AB, src_all, dst_all, w2, b2, nch):
    n, two_hid = AB.shape
    hid = two_hid // 2
    e = src_all.shape[0]
    epw = e // NW

    mesh = plsc.VectorSubcoreMesh(core_axis_name="c", subcore_axis_name="s")

    @functools.partial(
        pl.kernel,
        out_type=jax.ShapeDtypeStruct((e,), jnp.float32),
        mesh=mesh,
        compiler_params=pltpu.CompilerParams(needs_layout_passes=False),
        scratch_types=[
            pltpu.VMEM((hid,), jnp.float32),
            pltpu.VMEM((16,), jnp.float32),
        ]
        + [pltpu.VMEM((CH,), jnp.int32) for _ in range(RING)]
        + [pltpu.VMEM((CH,), jnp.int32) for _ in range(RING)]
        + [pltpu.VMEM((CH, two_hid), jnp.float32) for _ in range(RING)]
        + [pltpu.VMEM((CH, two_hid), jnp.float32) for _ in range(RING)]
        + [pltpu.VMEM((CH,), jnp.float32)]
        + [pltpu.SemaphoreType.DMA for _ in range(2 * RING)],
    )
    def k(ab_hbm, src_hbm, dst_hbm, w2_hbm, b2_hbm, out_hbm,
          w2v, b2v, *bufs):
        sv = bufs[0:RING]
        dvv = bufs[RING:2 * RING]
        arows = bufs[2 * RING:3 * RING]
        brows = bufs[3 * RING:4 * RING]
        outv = bufs[4 * RING]
        sa = bufs[4 * RING + 1:5 * RING + 1]
        sb = bufs[5 * RING + 1:6 * RING + 1]
        c_ax = lax.axis_index("c")
        s_ax = lax.axis_index("s")
        wid = c_ax * NS + s_ax
        pltpu.sync_copy(w2_hbm, w2v)
        pltpu.sync_copy(b2_hbm, b2v)
        w2r = [w2v[pl.ds(r * LN, LN)] for r in range(hid // LN)]
        b2vec = b2v[pl.ds(0, LN)]
        lane = lax.iota(jnp.int32, LN)

        def issue(cc, b):
            base = wid * epw + cc * CH
            pltpu.sync_copy(src_hbm.at[pl.ds(base, CH)], sv[b])
            pltpu.sync_copy(dst_hbm.at[pl.ds(base, CH)], dvv[b])
            pltpu.async_copy(ab_hbm.at[sv[b]], arows[b], sa[b])
            pltpu.async_copy(ab_hbm.at[dvv[b]], brows[b], sb[b])

        issue(0, 0)
        issue(1, 1)

        def outer(g, carry):
            for b in range(RING):
                cc = g * RING + b
                base = wid * epw + cc * CH
                pltpu.make_async_copy(ab_hbm.at[sv[b]], arows[b],
                                      sa[b]).wait()
                pltpu.make_async_copy(ab_hbm.at[dvv[b]], brows[b],
                                      sb[b]).wait()

                def group(gg, _):
                    acc = b2vec
                    for j in range(LN):
                        i = gg * LN + j
                        t = None
                        for r in range(hid // LN):
                            v = jnp.maximum(
                                arows[b][i, pl.ds(r * LN, LN)]
                                + brows[b][i, pl.ds(hid + r * LN, LN)],
                                0.0,
                            ) * w2r[r]
                            t = v if t is None else t + v
                        acc = jnp.where(lane == j, acc + jnp.sum(t), acc)
                    outv[pl.ds(gg * LN, LN)] = acc
                    return _

                lax.fori_loop(0, CH // LN, group, 0)
                pltpu.sync_copy(outv, out_hbm.at[pl.ds(base, CH)])

                @pl.when(cc + 2 < nch)
                def _():
                    issue(cc + 2, b)
            return carry

        lax.fori_loop(0, nch // RING, outer, 0)

    return k(AB, src_all, dst_all, w2, b2)


def kernel(batch, x, edge_index, beta, edge_attr, edge_weight,
           W_enc, b_enc, W1, b1, W2, b2):
    n, d = x.shape
    e = edge_index.shape[1]
    src = edge_index[0]
    dst = edge_index[1]

    np_ = ((n + NS * 8 - 1) // (NS * 8)) * (NS * 8)
    x = jnp.pad(x, ((0, np_ - n), (0, 0)))

    nch = -(-e // (NW * CH))
    nch = ((nch + RING - 1) // RING) * RING
    ep = nch * CH * NW
    src_p = jnp.pad(src, (0, ep - e))
    dst_p = jnp.pad(dst, (0, ep - e))
    ew_p = jnp.pad(edge_weight, (0, ep - e))

    h = _tc_encode(x, W_enc, b_enc)
    zeros = jnp.zeros((np_, d), dtype=jnp.float32)
    partials = _sc_aggregate(h, src_p, dst_p, ew_p, zeros, nch)
    AB = _tc_node_mlp(partials, h, beta, W1[:d], W1[d:], b1)
    b2pad = jnp.full((16,), b2[0], jnp.float32)
    logits = _sc_edge_logits(AB, src_p, dst_p, W2[:, 0], b2pad, nch)
    return logits[:e].reshape(e, 1)


# speedup vs baseline: 1.2669x; 1.2669x over previous
import functools

import jax
import jax.numpy as jnp
from jax import lax
from jax.experimental import pallas as pl
from jax.experimental.pallas import tpu as pltpu
from jax.experimental.pallas import tpu_sc as plsc

NC = 2
NS = 16
LN = 16
NW = NC * NS

CH = 80
RING = 2


def _tc_encode(x, W_enc, b_enc):
    def body(x_ref, w_ref, b_ref, o_ref):
        o_ref[...] = (
            jnp.dot(x_ref[...], w_ref[...], preferred_element_type=jnp.float32)
            + b_ref[...]
        )

    return pl.pallas_call(
        body,
        out_shape=jax.ShapeDtypeStruct(x.shape, jnp.float32),
    )(x, W_enc, b_enc.reshape(1, -1))


def _tc_node_mlp(p, h, beta, W1a, W1b, b1):
    n, d = h.shape
    hid = W1a.shape[1]

    def body(p_ref, h_ref, beta_ref, wa_ref, wb_ref, b1_ref, ab_ref):
        ne = jnp.maximum(p_ref[0] + p_ref[1] + beta_ref[0] * h_ref[...], 0.0)
        a = jnp.dot(ne, wa_ref[...], preferred_element_type=jnp.float32) + b1_ref[...]
        b = jnp.dot(ne, wb_ref[...], preferred_element_type=jnp.float32)
        ab_ref[...] = jnp.concatenate([a, b], axis=1)

    return pl.pallas_call(
        body,
        in_specs=[
            pl.BlockSpec(memory_space=pltpu.VMEM),
            pl.BlockSpec(memory_space=pltpu.VMEM),
            pl.BlockSpec(memory_space=pltpu.SMEM),
            pl.BlockSpec(memory_space=pltpu.VMEM),
            pl.BlockSpec(memory_space=pltpu.VMEM),
            pl.BlockSpec(memory_space=pltpu.VMEM),
        ],
        out_shape=jax.ShapeDtypeStruct((n, 2 * hid), jnp.float32),
    )(p, h, beta, W1a, W1b, b1.reshape(1, -1))


def _sc_aggregate(h, src_all, didx_all, ew_all, zeros, nch):
    n, d = h.shape
    epw = src_all.shape[0] // NW

    mesh = plsc.VectorSubcoreMesh(core_axis_name="c", subcore_axis_name="s")

    @functools.partial(
        pl.kernel,
        out_type=jax.ShapeDtypeStruct((NC, n, d), jnp.float32),
        mesh=mesh,
        compiler_params=pltpu.CompilerParams(needs_layout_passes=False),
        scratch_types=[
            pltpu.VMEM_SHARED((n, d), jnp.float32),
        ]
        + [pltpu.VMEM((CH,), jnp.int32) for _ in range(RING)]
        + [pltpu.VMEM((CH,), jnp.int32) for _ in range(RING)]
        + [pltpu.VMEM((CH,), jnp.float32) for _ in range(RING)]
        + [pltpu.VMEM((CH, d), jnp.float32) for _ in range(RING)]
        + [pltpu.SemaphoreType.DMA for _ in range(RING)],
    )
    def k(h_hbm, src_hbm, didx_hbm, ew_hbm, z_hbm, part_hbm, acc_sh, *bufs):
        sv = bufs[0:RING]
        dv = bufs[RING:2 * RING]
        ev = bufs[2 * RING:3 * RING]
        rows = bufs[3 * RING:4 * RING]
        gs = bufs[4 * RING:5 * RING]
        c_ax = lax.axis_index("c")
        s_ax = lax.axis_index("s")
        wid = c_ax * NS + s_ax
        slab = n // NS
        row0 = s_ax * slab
        pltpu.sync_copy(z_hbm.at[pl.ds(row0, slab)],
                        acc_sh.at[pl.ds(row0, slab)])
        plsc.subcore_barrier()

        def issue(cc, b):
            base = wid * epw + cc * CH
            pltpu.sync_copy(src_hbm.at[pl.ds(base, CH)], sv[b])
            pltpu.sync_copy(didx_hbm.at[pl.ds(base, CH)], dv[b])
            pltpu.sync_copy(ew_hbm.at[pl.ds(base, CH)], ev[b])
            pltpu.async_copy(h_hbm.at[sv[b]], rows[b], gs[b])

        issue(0, 0)
        issue(1, 1)

        def process(b):
            pltpu.make_async_copy(h_hbm.at[sv[b]], rows[b], gs[b]).wait()

            def scale(i, _):
                splat = jnp.zeros((LN,), jnp.int32) + i
                w = plsc.load_gather(ev[b], [splat])
                for r in range(d // LN):
                    rows[b][i, pl.ds(r * LN, LN)] = (
                        rows[b][i, pl.ds(r * LN, LN)] * w
                    )
                return _

            lax.fori_loop(0, CH, scale, 0, unroll=2)
            pltpu.sync_copy(rows[b], acc_sh.at[dv[b]], add=True)

        def outer(g, carry):
            for b in range(RING):
                cc = g * RING + b
                process(b)
                issue(cc + 2, b)
            return carry

        lax.fori_loop(0, (nch - 2) // RING, outer, 0)
        for b in range(RING):
            process(b)
        plsc.subcore_barrier()
        pltpu.sync_copy(acc_sh.at[pl.ds(row0, slab)],
                        part_hbm.at[c_ax, pl.ds(row0, slab)])

    return k(h, src_all, didx_all, ew_all, zeros)


def _sc_edge_logits(AB, src_all, dst_all, w2, b2, nch):
    n, two_hid = AB.shape
    hid = two_hid // 2
    e = src_all.shape[0]
    epw = e // NW

    mesh = plsc.VectorSubcoreMesh(core_axis_name="c", subcore_axis_name="s")

    @functools.partial(
        pl.kernel,
        out_type=jax.ShapeDtypeStruct((e,), jnp.float32),
        mesh=mesh,
        compiler_params=pltpu.CompilerParams(needs_layout_passes=False),
        scratch_types=[
            pltpu.VMEM((hid,), jnp.float32),
            pltpu.VMEM((16,), jnp.float32),
        ]
        + [pltpu.VMEM((CH,), jnp.int32) for _ in range(RING)]
        + [pltpu.VMEM((CH,), jnp.int32) for _ in range(RING)]
        + [pltpu.VMEM((CH, two_hid), jnp.float32) for _ in range(RING)]
        + [pltpu.VMEM((CH, two_hid), jnp.float32) for _ in range(RING)]
        + [pltpu.VMEM((CH,), jnp.float32)]
        + [pltpu.SemaphoreType.DMA for _ in range(2 * RING)],
    )
    def k(ab_hbm, src_hbm, dst_hbm, w2_hbm, b2_hbm, out_hbm,
          w2v, b2v, *bufs):
        sv = bufs[0:RING]
        dvv = bufs[RING:2 * RING]
        arows = bufs[2 * RING:3 * RING]
        brows = bufs[3 * RING:4 * RING]
        outv = bufs[4 * RING]
        sa = bufs[4 * RING + 1:5 * RING + 1]
        sb = bufs[5 * RING + 1:6 * RING + 1]
        c_ax = lax.axis_index("c")
        s_ax = lax.axis_index("s")
        wid = c_ax * NS + s_ax
        pltpu.sync_copy(w2_hbm, w2v)
        pltpu.sync_copy(b2_hbm, b2v)
        w2r = [w2v[pl.ds(r * LN, LN)] for r in range(hid // LN)]
        b2vec = b2v[pl.ds(0, LN)]
        lane = lax.iota(jnp.int32, LN)

        def issue(cc, b):
            base = wid * epw + cc * CH
            pltpu.sync_copy(src_hbm.at[pl.ds(base, CH)], sv[b])
            pltpu.sync_copy(dst_hbm.at[pl.ds(base, CH)], dvv[b])
            pltpu.async_copy(ab_hbm.at[sv[b]], arows[b], sa[b])
            pltpu.async_copy(ab_hbm.at[dvv[b]], brows[b], sb[b])

        issue(0, 0)
        issue(1, 1)

        def process(cc, b):
            base = wid * epw + cc * CH
            pltpu.make_async_copy(ab_hbm.at[sv[b]], arows[b], sa[b]).wait()
            pltpu.make_async_copy(ab_hbm.at[dvv[b]], brows[b], sb[b]).wait()

            def group(gg, _):
                acc = b2vec
                for j in range(LN):
                    i = gg * LN + j
                    t = None
                    for r in range(hid // LN):
                        v = jnp.maximum(
                            arows[b][i, pl.ds(r * LN, LN)]
                            + brows[b][i, pl.ds(hid + r * LN, LN)],
                            0.0,
                        ) * w2r[r]
                        t = v if t is None else t + v
                    acc = jnp.where(lane == j, acc + jnp.sum(t), acc)
                outv[pl.ds(gg * LN, LN)] = acc
                return _

            lax.fori_loop(0, CH // LN, group, 0)
            pltpu.sync_copy(outv, out_hbm.at[pl.ds(base, CH)])

        def outer(g, carry):
            for b in range(RING):
                cc = g * RING + b
                process(cc, b)
                issue(cc + 2, b)
            return carry

        lax.fori_loop(0, (nch - 2) // RING, outer, 0)
        for b in range(RING):
            process(nch - 2 + b, b)

    return k(AB, src_all, dst_all, w2, b2)


def kernel(batch, x, edge_index, beta, edge_attr, edge_weight,
           W_enc, b_enc, W1, b1, W2, b2):
    n, d = x.shape
    e = edge_index.shape[1]
    src = edge_index[0]
    dst = edge_index[1]

    np_ = ((n + NS * 8 - 1) // (NS * 8)) * (NS * 8)
    x = jnp.pad(x, ((0, np_ - n), (0, 0)))

    nch = -(-e // (NW * CH))
    nch = ((nch + RING - 1) // RING) * RING
    ep = nch * CH * NW
    src_p = jnp.pad(src, (0, ep - e))
    dst_p = jnp.pad(dst, (0, ep - e))
    ew_p = jnp.pad(edge_weight, (0, ep - e))

    h = _tc_encode(x, W_enc, b_enc)
    zeros = jnp.zeros((np_, d), dtype=jnp.float32)
    partials = _sc_aggregate(h, src_p, dst_p, ew_p, zeros, nch)
    AB = _tc_node_mlp(partials, h, beta, W1[:d], W1[d:], b1)
    b2pad = jnp.full((16,), b2[0], jnp.float32)
    logits = _sc_edge_logits(AB, src_p, dst_p, W2[:, 0], b2pad, nch)
    return logits[:e].reshape(e, 1)


# speedup vs baseline: 1.4809x; 1.1689x over previous
import functools

import jax
import jax.numpy as jnp
from jax import lax
from jax.experimental import pallas as pl
from jax.experimental.pallas import tpu as pltpu
from jax.experimental.pallas import tpu_sc as plsc

NC = 2
NS = 16
LN = 16
NW = NC * NS

CH = 80
RING = 2


def _tc_encode(x, W_enc, b_enc):
    def body(x_ref, w_ref, b_ref, o_ref):
        o_ref[...] = (
            jnp.dot(x_ref[...], w_ref[...], preferred_element_type=jnp.float32)
            + b_ref[...]
        )

    return pl.pallas_call(
        body,
        out_shape=jax.ShapeDtypeStruct(x.shape, jnp.float32),
    )(x, W_enc, b_enc.reshape(1, -1))


def _tc_node_mlp(p, h, beta, W1a, W1b, b1):
    n, d = h.shape
    hid = W1a.shape[1]

    def body(p_ref, h_ref, beta_ref, wa_ref, wb_ref, b1_ref, ab_ref):
        ne = jnp.maximum(p_ref[0] + p_ref[1] + beta_ref[0] * h_ref[...], 0.0)
        a = jnp.dot(ne, wa_ref[...], preferred_element_type=jnp.float32) + b1_ref[...]
        b = jnp.dot(ne, wb_ref[...], preferred_element_type=jnp.float32)
        ab_ref[...] = jnp.concatenate([a, b], axis=1)

    return pl.pallas_call(
        body,
        in_specs=[
            pl.BlockSpec(memory_space=pltpu.VMEM),
            pl.BlockSpec(memory_space=pltpu.VMEM),
            pl.BlockSpec(memory_space=pltpu.SMEM),
            pl.BlockSpec(memory_space=pltpu.VMEM),
            pl.BlockSpec(memory_space=pltpu.VMEM),
            pl.BlockSpec(memory_space=pltpu.VMEM),
        ],
        out_shape=jax.ShapeDtypeStruct((n, 2 * hid), jnp.float32),
    )(p, h, beta, W1a, W1b, b1.reshape(1, -1))


def _sc_aggregate(h, meta, zeros, nch):
    n, d = h.shape
    epw = meta.shape[0] * CH // NW

    mesh = plsc.VectorSubcoreMesh(core_axis_name="c", subcore_axis_name="s")

    @functools.partial(
        pl.kernel,
        out_type=jax.ShapeDtypeStruct((NC, n, d), jnp.float32),
        mesh=mesh,
        compiler_params=pltpu.CompilerParams(needs_layout_passes=False),
        scratch_types=[
            pltpu.VMEM_SHARED((n, d), jnp.float32),
        ]
        + [pltpu.VMEM((3, CH), jnp.int32) for _ in range(RING)]
        + [pltpu.VMEM((CH,), jnp.int32) for _ in range(RING)]
        + [pltpu.VMEM((CH,), jnp.int32) for _ in range(RING)]
        + [pltpu.VMEM((CH,), jnp.float32) for _ in range(RING)]
        + [pltpu.VMEM((CH, d), jnp.float32) for _ in range(RING)]
        + [pltpu.SemaphoreType.DMA for _ in range(RING)],
    )
    def k(h_hbm, meta_hbm, z_hbm, part_hbm, acc_sh, *bufs):
        mv = bufs[0:RING]
        sv = bufs[RING:2 * RING]
        dv = bufs[2 * RING:3 * RING]
        ev = bufs[3 * RING:4 * RING]
        rows = bufs[4 * RING:5 * RING]
        gs = bufs[5 * RING:6 * RING]
        c_ax = lax.axis_index("c")
        s_ax = lax.axis_index("s")
        wid = c_ax * NS + s_ax
        slab = n // NS
        row0 = s_ax * slab
        pltpu.sync_copy(z_hbm.at[pl.ds(row0, slab)],
                        acc_sh.at[pl.ds(row0, slab)])
        plsc.subcore_barrier()

        def issue(cc, b):
            pltpu.sync_copy(meta_hbm.at[wid * nch + cc], mv[b])
            for q in range(CH // LN):
                sl = pl.ds(q * LN, LN)
                sv[b][sl] = mv[b][0, sl]
            pltpu.async_copy(h_hbm.at[sv[b]], rows[b], gs[b])
            for q in range(CH // LN):
                sl = pl.ds(q * LN, LN)
                dv[b][sl] = mv[b][1, sl]
                ev[b][sl] = plsc.bitcast(mv[b][2, sl], jnp.float32)

        issue(0, 0)
        issue(1, 1)

        def process(b):
            pltpu.make_async_copy(h_hbm.at[sv[b]], rows[b], gs[b]).wait()

            def scale(i, _):
                splat = jnp.zeros((LN,), jnp.int32) + i
                w = plsc.load_gather(ev[b], [splat])
                for r in range(d // LN):
                    rows[b][i, pl.ds(r * LN, LN)] = (
                        rows[b][i, pl.ds(r * LN, LN)] * w
                    )
                return _

            lax.fori_loop(0, CH, scale, 0, unroll=2)
            pltpu.sync_copy(rows[b], acc_sh.at[dv[b]], add=True)

        def outer(g, carry):
            for b in range(RING):
                cc = g * RING + b
                process(b)
                issue(cc + 2, b)
            return carry

        lax.fori_loop(0, (nch - 2) // RING, outer, 0)
        for b in range(RING):
            process(b)
        plsc.subcore_barrier()
        pltpu.sync_copy(acc_sh.at[pl.ds(row0, slab)],
                        part_hbm.at[c_ax, pl.ds(row0, slab)])

    return k(h, meta, zeros)


def _sc_edge_logits(AB, meta, w2, b2, nch):
    n, two_hid = AB.shape
    hid = two_hid // 2
    e = meta.shape[0] * CH
    epw = e // NW

    mesh = plsc.VectorSubcoreMesh(core_axis_name="c", subcore_axis_name="s")

    @functools.partial(
        pl.kernel,
        out_type=jax.ShapeDtypeStruct((e,), jnp.float32),
        mesh=mesh,
        compiler_params=pltpu.CompilerParams(needs_layout_passes=False),
        scratch_types=[
            pltpu.VMEM((hid,), jnp.float32),
            pltpu.VMEM((16,), jnp.float32),
        ]
        + [pltpu.VMEM((2, CH), jnp.int32) for _ in range(RING)]
        + [pltpu.VMEM((CH,), jnp.int32) for _ in range(RING)]
        + [pltpu.VMEM((CH,), jnp.int32) for _ in range(RING)]
        + [pltpu.VMEM((CH, two_hid), jnp.float32) for _ in range(RING)]
        + [pltpu.VMEM((CH, two_hid), jnp.float32) for _ in range(RING)]
        + [pltpu.VMEM((CH,), jnp.float32)]
        + [pltpu.SemaphoreType.DMA for _ in range(2 * RING)],
    )
    def k(ab_hbm, meta_hbm, w2_hbm, b2_hbm, out_hbm,
          w2v, b2v, *bufs):
        mv = bufs[0:RING]
        sv = bufs[RING:2 * RING]
        dvv = bufs[2 * RING:3 * RING]
        arows = bufs[3 * RING:4 * RING]
        brows = bufs[4 * RING:5 * RING]
        outv = bufs[5 * RING]
        sa = bufs[5 * RING + 1:6 * RING + 1]
        sb = bufs[6 * RING + 1:7 * RING + 1]
        c_ax = lax.axis_index("c")
        s_ax = lax.axis_index("s")
        wid = c_ax * NS + s_ax
        pltpu.sync_copy(w2_hbm, w2v)
        pltpu.sync_copy(b2_hbm, b2v)
        w2r = [w2v[pl.ds(r * LN, LN)] for r in range(hid // LN)]
        b2vec = b2v[pl.ds(0, LN)]
        lane = lax.iota(jnp.int32, LN)

        def issue(cc, b):
            pltpu.sync_copy(meta_hbm.at[wid * nch + cc], mv[b])
            for q in range(CH // LN):
                sl = pl.ds(q * LN, LN)
                sv[b][sl] = mv[b][0, sl]
                dvv[b][sl] = mv[b][1, sl]
            pltpu.async_copy(ab_hbm.at[sv[b]], arows[b], sa[b])
            pltpu.async_copy(ab_hbm.at[dvv[b]], brows[b], sb[b])

        issue(0, 0)
        issue(1, 1)

        def process(cc, b):
            base = wid * epw + cc * CH
            pltpu.make_async_copy(ab_hbm.at[sv[b]], arows[b], sa[b]).wait()
            pltpu.make_async_copy(ab_hbm.at[dvv[b]], brows[b], sb[b]).wait()

            def group(gg, _):
                acc = b2vec
                for j in range(LN):
                    i = gg * LN + j
                    t = None
                    for r in range(hid // LN):
                        v = jnp.maximum(
                            arows[b][i, pl.ds(r * LN, LN)]
                            + brows[b][i, pl.ds(hid + r * LN, LN)],
                            0.0,
                        ) * w2r[r]
                        t = v if t is None else t + v
                    acc = jnp.where(lane == j, acc + jnp.sum(t), acc)
                outv[pl.ds(gg * LN, LN)] = acc
                return _

            lax.fori_loop(0, CH // LN, group, 0)
            pltpu.sync_copy(outv, out_hbm.at[pl.ds(base, CH)])

        def outer(g, carry):
            for b in range(RING):
                cc = g * RING + b
                process(cc, b)
                issue(cc + 2, b)
            return carry

        lax.fori_loop(0, (nch - 2) // RING, outer, 0)
        for b in range(RING):
            process(nch - 2 + b, b)

    return k(AB, meta, w2, b2)


def kernel(batch, x, edge_index, beta, edge_attr, edge_weight,
           W_enc, b_enc, W1, b1, W2, b2):
    n, d = x.shape
    e = edge_index.shape[1]
    src = edge_index[0]
    dst = edge_index[1]

    np_ = ((n + NS * 8 - 1) // (NS * 8)) * (NS * 8)
    x = jnp.pad(x, ((0, np_ - n), (0, 0)))

    nch = -(-e // (NW * CH))
    nch = ((nch + RING - 1) // RING) * RING
    ep = nch * CH * NW
    src_p = jnp.pad(src, (0, ep - e)).reshape(-1, CH)
    dst_p = jnp.pad(dst, (0, ep - e)).reshape(-1, CH)
    ew_p = lax.bitcast_convert_type(
        jnp.pad(edge_weight, (0, ep - e)), jnp.int32).reshape(-1, CH)
    meta2 = jnp.stack([src_p, dst_p, ew_p], axis=1)
    meta4 = jnp.stack([src_p, dst_p], axis=1)

    h = _tc_encode(x, W_enc, b_enc)
    zeros = jnp.zeros((np_, d), dtype=jnp.float32)
    partials = _sc_aggregate(h, meta2, zeros, nch)
    AB = _tc_node_mlp(partials, h, beta, W1[:d], W1[d:], b1)
    b2pad = jnp.full((16,), b2[0], jnp.float32)
    logits = _sc_edge_logits(AB, meta4, W2[:, 0], b2pad, nch)
    return logits[:e].reshape(e, 1)
